# baseline (speedup 1.0000x reference)
import functools

import jax
import jax.numpy as jnp
from jax import lax
from jax.experimental import pallas as pl
from jax.experimental.pallas import tpu as pltpu
from jax.experimental.pallas import tpu_sc as plsc

_NUM_CLASSES = 6
_KDIM = 128
_NC = 2
_NS = 16
_NW = _NC * _NS
_CHUNK = 98304
_NCHUNK = 4


def _q_kernel(pr_ref, wq_ref, bq_ref, q_ref):
    for i in range(_NUM_CLASSES):
        for j in range(2):
            p = pr_ref[:, i, j, :]
            q = jax.lax.dot_general(
                p, wq_ref[i], (((1,), (1,)), ((), ())),
                preferred_element_type=jnp.float32,
            )
            q_ref[i * 2 + j] = q + bq_ref[i][None, :]


def _sc_scale(x_hbm, o_hbm, buf):
    wid = lax.axis_index("c") * _NS + lax.axis_index("s")
    for k in range(_NCHUNK):
        off = (wid * _NCHUNK + k) * _CHUNK
        pltpu.sync_copy(x_hbm.at[pl.ds(off, _CHUNK)], buf)

        def body(i, carry):
            sl = pl.ds(i * 16, 16)
            buf[sl] = buf[sl] * jnp.float32(1.0 / 12.0)
            return carry

        lax.fori_loop(0, _CHUNK // 16, body, 0)
        pltpu.sync_copy(buf, o_hbm.at[pl.ds(off, _CHUNK)])


def kernel(prototypes, assp_features, DomainTrain, Wk, bk, Wq, bq):
    b, c, h, w = assp_features.shape
    nc = prototypes.shape[1]
    pn = prototypes.shape[2]
    npairs = nc * pn
    n = b * c * h * w

    q_all = pl.pallas_call(
        _q_kernel,
        out_shape=jax.ShapeDtypeStruct((npairs, b, _KDIM), jnp.float32),
    )(prototypes, Wq, bq)

    xf = jnp.transpose(assp_features, (0, 2, 3, 1)).reshape(n)

    mesh = plsc.VectorSubcoreMesh(core_axis_name="c", subcore_axis_name="s")
    sc_scale = functools.partial(
        pl.kernel,
        out_type=jax.ShapeDtypeStruct((n,), jnp.float32),
        mesh=mesh,
        scratch_types=[pltpu.VMEM((_CHUNK,), jnp.float32)],
    )(_sc_scale)
    of = sc_scale(xf)

    out = jnp.transpose(of.reshape(b, h, w, c), (0, 3, 1, 2))
    return (out,) + tuple(q_all[p] for p in range(npairs))


# speedup vs baseline: 8.4713x; 8.4713x over previous
import jax
import jax.numpy as jnp
from jax.experimental import pallas as pl

_NUM_CLASSES = 6
_KDIM = 128
_BH = 64
_BB = 2


def _fused_kernel(pr_ref, wq_ref, bq_ref, x_ref, o_ref, q_ref):
    o_ref[...] = x_ref[...] * jnp.float32(1.0 / 12.0)

    b = pl.program_id(0)
    hblk = pl.program_id(1)

    @pl.when(jnp.logical_and(b == 0, hblk == 0))
    def _():
        for i in range(_NUM_CLASSES):
            for j in range(2):
                p = pr_ref[:, i, j, :]
                q = jax.lax.dot_general(
                    p, wq_ref[i], (((1,), (1,)), ((), ())),
                    preferred_element_type=jnp.float32,
                )
                q_ref[i * 2 + j] = q + bq_ref[i][None, :]


def kernel(prototypes, assp_features, DomainTrain, Wk, bk, Wq, bq):
    b, c, h, w = assp_features.shape
    nc = prototypes.shape[1]
    pn = prototypes.shape[2]
    npairs = nc * pn

    xt = jnp.transpose(assp_features, (0, 2, 3, 1))
    grid = (b // _BB, h // _BH)
    out_t, q_all = pl.pallas_call(
        _fused_kernel,
        grid=grid,
        in_specs=[
            pl.BlockSpec(prototypes.shape, lambda bi, hi: (0, 0, 0, 0)),
            pl.BlockSpec(Wq.shape, lambda bi, hi: (0, 0, 0)),
            pl.BlockSpec(bq.shape, lambda bi, hi: (0, 0)),
            pl.BlockSpec((_BB, _BH, w, c), lambda bi, hi: (bi, hi, 0, 0)),
        ],
        out_specs=[
            pl.BlockSpec((_BB, _BH, w, c), lambda bi, hi: (bi, hi, 0, 0)),
            pl.BlockSpec((npairs, b, _KDIM), lambda bi, hi: (0, 0, 0)),
        ],
        out_shape=[
            jax.ShapeDtypeStruct((b, h, w, c), jnp.float32),
            jax.ShapeDtypeStruct((npairs, b, _KDIM), jnp.float32),
        ],
    )(prototypes, Wq, bq, xt)

    out = jnp.transpose(out_t, (0, 3, 1, 2))
    return (out,) + tuple(q_all[p] for p in range(npairs))
